# baseline (speedup 1.0000x reference)
import jax
import jax.numpy as jnp
from jax import lax
from jax.experimental import pallas as pl
from jax.experimental.pallas import tpu as pltpu
from jax.experimental.pallas import tpu_sc as plsc

NF = 26
V = 100000
B = 16384
NC, NS, L = 2, 16, 16
HB = B // 2
RPT = HB // NS
TCH = 128
GCH = 1024
RCH = 128

_CP = pltpu.CompilerParams(needs_layout_passes=False)
_MESH = dict(core_axis_name="c", subcore_axis_name="s",
             num_cores=NC, num_subcores=NS)


def _main_body(xflat_hbm, t2d_hbm, out2_hbm,
               xT_sh, trow, bufA, bufB, idxc, red, outq, sem):
    c_id = lax.axis_index("c")
    s_id = lax.axis_index("s")
    wid = s_id * NC + c_id
    iota = lax.iota(jnp.int32, L)
    vec26 = iota * NF

    @pl.when(wid < NF)
    def _():
        pltpu.async_copy(t2d_hbm.at[wid], trow, sem)

    for h in range(2):
        def chunk_a(k, _):
            r0 = h * HB + s_id * RPT + k * TCH
            pltpu.sync_copy(xflat_hbm.at[pl.ds(r0 * NF, TCH * NF)], bufA)

            def frow(f, _):
                for j in range(TCH // L):
                    p = vec26 + (j * (L * NF) + f)
                    g = plsc.load_gather(bufA, [p])
                    bufB[f, pl.ds(j * L, L)] = plsc.bitcast(g, jnp.float32)
                return 0
            lax.fori_loop(0, NF, frow, 0)
            pltpu.sync_copy(bufB, xT_sh.at[:, pl.ds(s_id * RPT + k * TCH, TCH)])
            return 0
        lax.fori_loop(0, RPT // TCH, chunk_a, 0)
        plsc.subcore_barrier()

        @pl.when(wid < NF)
        def _():
            if h == 0:
                pltpu.make_async_copy(t2d_hbm.at[wid], trow, sem).wait()

            def chunk_b(cb, _):
                lbase = cb * GCH
                pltpu.sync_copy(xT_sh.at[wid, pl.ds(lbase, GCH)], idxc)

                def g8(i, _):
                    for j in range(8):
                        off = i * (8 * L) + j * L
                        ids = plsc.bitcast(idxc[pl.ds(off, L)], jnp.int32)
                        idxc[pl.ds(off, L)] = plsc.load_gather(trow, [ids])
                    return 0
                lax.fori_loop(0, GCH // (8 * L), g8, 0)
                pltpu.sync_copy(idxc, xT_sh.at[wid, pl.ds(lbase, GCH)])
                return 0
            lax.fori_loop(0, HB // GCH, chunk_b, 0)
        plsc.subcore_barrier()

        def chunk_c(q, _):
            col0 = s_id * RPT + q * RCH
            pltpu.sync_copy(xT_sh.at[:, pl.ds(col0, RCH)], red)

            def rv(v, _):
                acc = red[c_id, pl.ds(v * L, L)]
                for j in range(1, 13):
                    acc = acc + red[2 * j + c_id, pl.ds(v * L, L)]
                outq[pl.ds(q * RCH + v * L, L)] = acc
                return 0
            lax.fori_loop(0, RCH // L, rv, 0)
            return 0
        lax.fori_loop(0, RPT // RCH, chunk_c, 0)
        pltpu.sync_copy(outq, out2_hbm.at[c_id, pl.ds(h * HB + s_id * RPT, RPT)])
        if h == 0:
            plsc.subcore_barrier()


def _tc_add_body(p_ref, o_ref):
    o_ref[...] = p_ref[0] + p_ref[1]


def kernel(X, table):
    xflat = X.reshape(B * NF)
    t2d = table.reshape(NF, V)

    out2 = pl.kernel(
        _main_body,
        out_type=jax.ShapeDtypeStruct((NC, B), jnp.float32),
        mesh=plsc.VectorSubcoreMesh(**_MESH),
        scratch_types=[
            pltpu.VMEM_SHARED((32, HB), jnp.float32),
            pltpu.VMEM((V,), jnp.float32),
            pltpu.VMEM((TCH * NF,), jnp.int32),
            pltpu.VMEM((32, TCH), jnp.float32),
            pltpu.VMEM((GCH,), jnp.float32),
            pltpu.VMEM((32, RCH), jnp.float32),
            pltpu.VMEM((RPT,), jnp.float32),
            pltpu.SemaphoreType.DMA,
        ],
        compiler_params=_CP,
    )(xflat, t2d)

    out = pl.pallas_call(
        _tc_add_body,
        out_shape=jax.ShapeDtypeStruct((128, 128), jnp.float32),
    )(out2.reshape(NC, 128, 128))
    return out.reshape(B, 1)
